# baseline (speedup 1.0000x reference)
import jax
import jax.numpy as jnp
from jax.experimental import pallas as pl
from jax.experimental.pallas import tpu as pltpu

TN = 400


def _contract0(a, b):
    return jax.lax.dot_general(
        a, b, dimension_numbers=(((0,), (0,)), ((), ())),
        preferred_element_type=jnp.float32)


def _mm(a, b):
    return jnp.dot(a, b, preferred_element_type=jnp.float32)


def _fused_kernel(b_ref, x0_ref, w1_ref, w2_ref, x0_out_ref, x1_out_ref,
                  bbf_ref, acc_ref, h_ref):
    p = pl.program_id(0)
    j = pl.program_id(1)
    nt = pl.num_programs(1)
    tn = b_ref.shape[0]

    @pl.when(jnp.logical_and(p == 0, j == 0))
    def _():
        acc_ref[...] = jnp.zeros_like(acc_ref)

    @pl.when(p == 0)
    def _():
        b = b_ref[...].astype(jnp.bfloat16)
        bbf_ref[pl.ds(j * tn, tn), :] = b
        acc_ref[...] += _contract0(b, x0_ref[...].astype(jnp.bfloat16))

    @pl.when(jnp.logical_and(p == 0, j == nt - 1))
    def _():
        h_ref[...] = _mm(acc_ref[...].astype(jnp.bfloat16),
                         w1_ref[...].astype(jnp.bfloat16)).astype(jnp.bfloat16)
        acc_ref[...] = jnp.zeros_like(acc_ref)

    @pl.when(p == 1)
    def _():
        b = bbf_ref[pl.ds(j * tn, tn), :]
        x0b = _mm(b, h_ref[...])
        acc_ref[...] += _contract0(b, x0b.astype(jnp.bfloat16))

    @pl.when(jnp.logical_and(p == 1, j == nt - 1))
    def _():
        x1_out_ref[...] = acc_ref[...]
        h_ref[...] = _mm(acc_ref[...].astype(jnp.bfloat16),
                         w2_ref[...].astype(jnp.bfloat16)).astype(jnp.bfloat16)

    @pl.when(p == 2)
    def _():
        x0_out_ref[...] = _mm(bbf_ref[pl.ds(j * tn, tn), :], h_ref[...])


@jax.jit
def kernel(x_0, incidence_1, W1, W2):
    n, e = incidence_1.shape
    d = x_0.shape[1]
    nt = n // TN
    f32 = jnp.float32

    x0_out, x1_out = pl.pallas_call(
        _fused_kernel,
        grid=(3, nt),
        in_specs=[
            pl.BlockSpec((TN, e), lambda p, j: (jnp.where(p == 0, j, 0), 0)),
            pl.BlockSpec((TN, d), lambda p, j: (jnp.where(p == 0, j, 0), 0)),
            pl.BlockSpec((d, d), lambda p, j: (0, 0)),
            pl.BlockSpec((d, d), lambda p, j: (0, 0)),
        ],
        out_specs=[
            pl.BlockSpec((TN, d), lambda p, j: (jnp.where(p == 2, j, 0), 0)),
            pl.BlockSpec((e, d), lambda p, j: (0, 0)),
        ],
        out_shape=[
            jax.ShapeDtypeStruct((n, d), f32),
            jax.ShapeDtypeStruct((e, d), f32),
        ],
        scratch_shapes=[
            pltpu.VMEM((n, e), jnp.bfloat16),
            pltpu.VMEM((e, d), f32),
            pltpu.VMEM((e, d), jnp.bfloat16),
        ],
        compiler_params=pltpu.CompilerParams(
            dimension_semantics=("arbitrary", "arbitrary")),
    )(incidence_1, x_0, W1, W2)

    return (x0_out, x1_out)


# speedup vs baseline: 1.0697x; 1.0697x over previous
import jax
import jax.numpy as jnp
from jax.experimental import pallas as pl
from jax.experimental.pallas import tpu as pltpu

TN = 400


def _contract0(a, b):
    return jax.lax.dot_general(
        a, b, dimension_numbers=(((0,), (0,)), ((), ())),
        preferred_element_type=jnp.float32)


def _mm(a, b):
    return jnp.dot(a, b, preferred_element_type=jnp.float32)


def _fused_kernel(b_ref, x0_ref, w1_ref, w2_ref, x0_out_ref, x1_out_ref,
                  bbf_ref, acct_ref, h_ref):
    p = pl.program_id(0)
    j = pl.program_id(1)
    nt = pl.num_programs(1)
    tn = b_ref.shape[0]

    @pl.when(jnp.logical_and(p == 0, j == 0))
    def _():
        acct_ref[...] = jnp.zeros_like(acct_ref)

    @pl.when(p == 0)
    def _():
        b = b_ref[...].astype(jnp.bfloat16)
        bbf_ref[pl.ds(j * tn, tn), :] = b
        acct_ref[...] += _contract0(x0_ref[...].astype(jnp.bfloat16), b)

    @pl.when(jnp.logical_and(p == 0, j == nt - 1))
    def _():
        x1 = jnp.swapaxes(acct_ref[...], 0, 1).astype(jnp.bfloat16)
        h_ref[...] = _mm(x1, w1_ref[...].astype(jnp.bfloat16)
                         ).astype(jnp.bfloat16)
        acct_ref[...] = jnp.zeros_like(acct_ref)

    @pl.when(p == 1)
    def _():
        b = bbf_ref[pl.ds(j * tn, tn), :]
        x0b = _mm(b, h_ref[...])
        acct_ref[...] += _contract0(x0b.astype(jnp.bfloat16), b)

    @pl.when(jnp.logical_and(p == 1, j == nt - 1))
    def _():
        x1p = jnp.swapaxes(acct_ref[...], 0, 1)
        x1_out_ref[...] = x1p
        h_ref[...] = _mm(x1p.astype(jnp.bfloat16),
                         w2_ref[...].astype(jnp.bfloat16)).astype(jnp.bfloat16)

    @pl.when(p == 2)
    def _():
        x0_out_ref[...] = _mm(bbf_ref[pl.ds(j * tn, tn), :], h_ref[...])


@jax.jit
def kernel(x_0, incidence_1, W1, W2):
    n, e = incidence_1.shape
    d = x_0.shape[1]
    nt = n // TN
    f32 = jnp.float32

    x0_out, x1_out = pl.pallas_call(
        _fused_kernel,
        grid=(3, nt),
        in_specs=[
            pl.BlockSpec((TN, e), lambda p, j: (jnp.where(p == 0, j, 0), 0)),
            pl.BlockSpec((TN, d), lambda p, j: (jnp.where(p == 0, j, 0), 0)),
            pl.BlockSpec((d, d), lambda p, j: (0, 0)),
            pl.BlockSpec((d, d), lambda p, j: (0, 0)),
        ],
        out_specs=[
            pl.BlockSpec((TN, d), lambda p, j: (jnp.where(p == 2, j, 0), 0)),
            pl.BlockSpec((e, d), lambda p, j: (0, 0)),
        ],
        out_shape=[
            jax.ShapeDtypeStruct((n, d), f32),
            jax.ShapeDtypeStruct((e, d), f32),
        ],
        scratch_shapes=[
            pltpu.VMEM((n, e), jnp.bfloat16),
            pltpu.VMEM((d, e), f32),
            pltpu.VMEM((e, d), jnp.bfloat16),
        ],
        compiler_params=pltpu.CompilerParams(
            dimension_semantics=("arbitrary", "arbitrary")),
    )(incidence_1, x_0, W1, W2)

    return (x0_out, x1_out)


# speedup vs baseline: 1.1710x; 1.0947x over previous
import jax
import jax.numpy as jnp
from jax.experimental import pallas as pl
from jax.experimental.pallas import tpu as pltpu

TN = 400
T2 = 1000


def _contract0(a, b):
    return jax.lax.dot_general(
        a, b, dimension_numbers=(((0,), (0,)), ((), ())),
        preferred_element_type=jnp.float32)


def _mm(a, b):
    return jnp.dot(a, b, preferred_element_type=jnp.float32)


def _fused_kernel(b_ref, x0_ref, w1_ref, w2_ref, x0_out_ref, x1_out_ref,
                  bbf_ref, acct_ref):
    j = pl.program_id(0)
    nt = pl.num_programs(0)
    tn = b_ref.shape[0]
    bf16 = jnp.bfloat16

    @pl.when(j == 0)
    def _():
        acct_ref[...] = jnp.zeros_like(acct_ref)

    b = b_ref[...].astype(bf16)
    bbf_ref[pl.ds(j * tn, tn), :] = b
    acct_ref[...] += _contract0(x0_ref[...].astype(bf16), b)

    @pl.when(j == nt - 1)
    def _():
        n2 = bbf_ref.shape[0] // T2
        x1 = jnp.swapaxes(acct_ref[...], 0, 1).astype(bf16)
        h1 = _mm(x1, w1_ref[...].astype(bf16)).astype(bf16)
        acct_ref[...] = jnp.zeros_like(acct_ref)

        def agg_body(i, _):
            bb = bbf_ref[pl.ds(i * T2, T2), :]
            x0b = _mm(bb, h1)
            acct_ref[...] += _contract0(x0b.astype(bf16), bb)
            return 0

        jax.lax.fori_loop(0, n2, agg_body, 0)

        x1p = jnp.swapaxes(acct_ref[...], 0, 1)
        x1_out_ref[...] = x1p
        h2 = _mm(x1p.astype(bf16), w2_ref[...].astype(bf16)).astype(bf16)

        def out_body(i, _):
            x0_out_ref[pl.ds(i * T2, T2), :] = _mm(
                bbf_ref[pl.ds(i * T2, T2), :], h2)
            return 0

        jax.lax.fori_loop(0, n2, out_body, 0)


@jax.jit
def kernel(x_0, incidence_1, W1, W2):
    n, e = incidence_1.shape
    d = x_0.shape[1]
    nt = n // TN
    f32 = jnp.float32

    x0_out, x1_out = pl.pallas_call(
        _fused_kernel,
        grid=(nt,),
        in_specs=[
            pl.BlockSpec((TN, e), lambda j: (j, 0)),
            pl.BlockSpec((TN, d), lambda j: (j, 0)),
            pl.BlockSpec((d, d), lambda j: (0, 0)),
            pl.BlockSpec((d, d), lambda j: (0, 0)),
        ],
        out_specs=[
            pl.BlockSpec((n, d), lambda j: (0, 0)),
            pl.BlockSpec((e, d), lambda j: (0, 0)),
        ],
        out_shape=[
            jax.ShapeDtypeStruct((n, d), f32),
            jax.ShapeDtypeStruct((e, d), f32),
        ],
        scratch_shapes=[
            pltpu.VMEM((n, e), jnp.bfloat16),
            pltpu.VMEM((d, e), f32),
        ],
        compiler_params=pltpu.CompilerParams(
            dimension_semantics=("arbitrary",)),
    )(incidence_1, x_0, W1, W2)

    return (x0_out, x1_out)


# speedup vs baseline: 1.1730x; 1.0017x over previous
import jax
import jax.numpy as jnp
from jax.experimental import pallas as pl
from jax.experimental.pallas import tpu as pltpu

TN = 400
T2 = 1000


def _contract0(a, b):
    return jax.lax.dot_general(
        a, b, dimension_numbers=(((0,), (0,)), ((), ())),
        preferred_element_type=jnp.float32)


def _mm(a, b):
    return jnp.dot(a, b, preferred_element_type=jnp.float32)


def _fused_kernel(b_ref, x0_ref, w1_ref, w2_ref, x0_out_ref, x1_out_ref,
                  bbf_ref, acct_ref):
    j = pl.program_id(0)
    nt = pl.num_programs(0)
    tn = b_ref.shape[0]
    bf16 = jnp.bfloat16

    @pl.when(j == 0)
    def _():
        acct_ref[...] = jnp.zeros_like(acct_ref)

    b = b_ref[...].astype(bf16)
    bbf_ref[pl.ds(j * tn, tn), :] = b
    acct_ref[...] += _contract0(x0_ref[...].astype(bf16), b)

    @pl.when(j == nt - 1)
    def _():
        n2 = bbf_ref.shape[0] // T2
        x1 = jnp.swapaxes(acct_ref[...], 0, 1).astype(bf16)
        h1 = _mm(x1, w1_ref[...].astype(bf16)).astype(bf16)

        acct_ref[...] = jnp.zeros_like(acct_ref)
        for i in range(n2):
            bb = bbf_ref[pl.ds(i * T2, T2), :]
            x0b = _mm(bb, h1)
            acct_ref[...] += _contract0(x0b.astype(bf16), bb)

        x1p = jnp.swapaxes(acct_ref[...], 0, 1)
        x1_out_ref[...] = x1p
        h2 = _mm(x1p.astype(bf16), w2_ref[...].astype(bf16)).astype(bf16)

        for i in range(n2):
            x0_out_ref[pl.ds(i * T2, T2), :] = _mm(
                bbf_ref[pl.ds(i * T2, T2), :], h2)


@jax.jit
def kernel(x_0, incidence_1, W1, W2):
    n, e = incidence_1.shape
    d = x_0.shape[1]
    nt = n // TN
    f32 = jnp.float32

    x0_out, x1_out = pl.pallas_call(
        _fused_kernel,
        grid=(nt,),
        in_specs=[
            pl.BlockSpec((TN, e), lambda j: (j, 0)),
            pl.BlockSpec((TN, d), lambda j: (j, 0)),
            pl.BlockSpec((d, d), lambda j: (0, 0)),
            pl.BlockSpec((d, d), lambda j: (0, 0)),
        ],
        out_specs=[
            pl.BlockSpec((n, d), lambda j: (0, 0)),
            pl.BlockSpec((e, d), lambda j: (0, 0)),
        ],
        out_shape=[
            jax.ShapeDtypeStruct((n, d), f32),
            jax.ShapeDtypeStruct((e, d), f32),
        ],
        scratch_shapes=[
            pltpu.VMEM((n, e), jnp.bfloat16),
            pltpu.VMEM((d, e), f32),
        ],
        compiler_params=pltpu.CompilerParams(
            dimension_semantics=("arbitrary",)),
    )(incidence_1, x_0, W1, W2)

    return (x0_out, x1_out)


# speedup vs baseline: 2.1703x; 1.8502x over previous
import jax
import jax.numpy as jnp
from jax.experimental import pallas as pl
from jax.experimental.pallas import tpu as pltpu

TC = 512


def _mm(a, b):
    return jnp.dot(a, b, preferred_element_type=jnp.float32)


def _make_kernel(n):
  def _fused_kernel(bt_ref, x0_ref, w1_ref, w2_ref, x0_out_ref, x1_out_ref,
                    bbf_ref, acc_ref, ht_ref):
    p = pl.program_id(0)
    j = pl.program_id(1)
    nt = pl.num_programs(1)
    e, tc = bt_ref.shape
    bf16 = jnp.bfloat16

    @pl.when(jnp.logical_and(p == 0, j == 0))
    def _():
        acc_ref[...] = jnp.zeros_like(acc_ref)

    @pl.when(p == 0)
    def _():
        b = bt_ref[...].astype(bf16)

        @pl.when(j == nt - 1)
        def _():
            rem = n - (nt - 1) * tc
            col = jax.lax.broadcasted_iota(jnp.int32, (e, tc), 1)
            bbf_ref[j] = jnp.where(col < rem, b, jnp.zeros_like(b))

        @pl.when(j != nt - 1)
        def _():
            bbf_ref[j] = b

        x0t = x0_ref[...].astype(bf16)
        rem = n - (nt - 1) * tc
        row = jax.lax.broadcasted_iota(jnp.int32, x0t.shape, 0)
        x0t = jnp.where(jnp.logical_or(j != nt - 1, row < rem), x0t,
                        jnp.zeros_like(x0t))
        acc_ref[...] += _mm(bbf_ref[j], x0t)

    @pl.when(jnp.logical_and(p == 0, j == nt - 1))
    def _():
        h1 = _mm(acc_ref[...].astype(bf16), w1_ref[...].astype(bf16))
        ht_ref[...] = jnp.swapaxes(h1, 0, 1).astype(bf16)
        acc_ref[...] = jnp.zeros_like(acc_ref)

    @pl.when(p == 1)
    def _():
        bb = bbf_ref[j]
        x0bt = _mm(ht_ref[...], bb)
        x0b = jnp.swapaxes(x0bt, 0, 1).astype(bf16)
        acc_ref[...] += _mm(bb, x0b)

    @pl.when(jnp.logical_and(p == 1, j == nt - 1))
    def _():
        x1p = acc_ref[...]
        x1_out_ref[...] = x1p
        h2 = _mm(x1p.astype(bf16), w2_ref[...].astype(bf16))
        ht_ref[...] = jnp.swapaxes(h2, 0, 1).astype(bf16)

    @pl.when(p == 2)
    def _():
        t = _mm(ht_ref[...], bbf_ref[j])
        x0_out_ref[...] = jnp.swapaxes(t, 0, 1)

  return _fused_kernel


@jax.jit
def kernel(x_0, incidence_1, W1, W2):
    n, e = incidence_1.shape
    d = x_0.shape[1]
    nt = (n + TC - 1) // TC
    f32 = jnp.float32
    bt = incidence_1.T

    x0_out, x1_out = pl.pallas_call(
        _make_kernel(n),
        grid=(3, nt),
        in_specs=[
            pl.BlockSpec((e, TC), lambda p, j: (0, jnp.where(p == 0, j, 0))),
            pl.BlockSpec((TC, d), lambda p, j: (jnp.where(p == 0, j, 0), 0)),
            pl.BlockSpec((d, d), lambda p, j: (0, 0)),
            pl.BlockSpec((d, d), lambda p, j: (0, 0)),
        ],
        out_specs=[
            pl.BlockSpec((TC, d), lambda p, j: (jnp.where(p == 2, j, 0), 0)),
            pl.BlockSpec((e, d), lambda p, j: (0, 0)),
        ],
        out_shape=[
            jax.ShapeDtypeStruct((n, d), f32),
            jax.ShapeDtypeStruct((e, d), f32),
        ],
        scratch_shapes=[
            pltpu.VMEM((nt, e, TC), jnp.bfloat16),
            pltpu.VMEM((e, d), f32),
            pltpu.VMEM((d, e), jnp.bfloat16),
        ],
        compiler_params=pltpu.CompilerParams(
            dimension_semantics=("arbitrary", "arbitrary")),
    )(bt, x_0, W1, W2)

    return (x0_out, x1_out)
